# baseline (speedup 1.0000x reference)
import jax
import jax.numpy as jnp
from jax import lax
from jax.experimental import pallas as pl
from jax.experimental.pallas import tpu as pltpu

N = 4096
C = 3
H = 256
W = 256

PIX_BLK = 1024
N_BLK = 512
N_ROWS = PIX_BLK // W


def _raster_body(xyz_ref, chol_ref, colors_ref, out_ref):
    p = pl.program_id(0)
    nb = pl.program_id(1)

    xy = jnp.tanh(xyz_ref[...])
    cx = 0.5 * W * (xy[0:1, :] + 1.0)
    cy = 0.5 * H * (xy[1:2, :] + 1.0)
    l11 = chol_ref[0:1, :]
    l21 = chol_ref[1:2, :]
    l22 = chol_ref[2:3, :]
    s11 = l11 * l11
    s12 = l11 * l21
    s22 = l21 * l21 + l22 * l22
    det = s11 * s22 - s12 * s12
    inv_det = 1.0 / det
    a = (0.5 * s22) * inv_det
    b = (-s12) * inv_det
    c = (0.5 * s11) * inv_det

    gx = -(2.0 * a * cx + b * cy)
    gy = -(b * cx + 2.0 * c * cy)
    g0 = (a * cx + b * cy) * cx + c * cy * cy
    grow = lax.broadcasted_iota(jnp.int32, (8, N_BLK), 0)
    zros = jnp.zeros((8, N_BLK), jnp.float32)

    def _sel8(pieces, idx, acc):
        for k, piece in enumerate(pieces):
            acc = jnp.where(idx == k, piece, acc)
        return acc

    gmat = _sel8([a, b, c, gx, gy, g0], grow, zros)

    lin = lax.broadcasted_iota(jnp.int32, (PIX_BLK, 1), 0)
    row0 = (p * N_ROWS).astype(jnp.float32)
    xs = (lin % W).astype(jnp.float32) + 0.5
    ys = (lin // W).astype(jnp.float32) + (row0 + 0.5)
    ones = jnp.ones_like(xs)
    pcol = lax.broadcasted_iota(jnp.int32, (PIX_BLK, 8), 1)
    phi = _sel8([xs * xs, xs * ys, ys * ys, xs, ys, ones], pcol,
                jnp.zeros((PIX_BLK, 8), jnp.float32))

    sigma = jnp.dot(phi, gmat, preferred_element_type=jnp.float32,
                    precision=lax.Precision.HIGHEST)
    wgt = jnp.exp(-jnp.maximum(sigma, 0.0))

    contrib = jnp.dot(wgt, colors_ref[...], preferred_element_type=jnp.float32)

    @pl.when(nb == 0)
    def _():
        out_ref[...] = contrib

    @pl.when(nb != 0)
    def _():
        out_ref[...] += contrib


def kernel(_xyz, _cholesky, _features_dc, _opacity, cholesky_bound, render_colors):
    colors = jnp.transpose(_features_dc, (1, 0, 2)).reshape(N, C * 3)
    colors = colors * _opacity[:, 0:1]
    colors = jnp.pad(colors, ((0, 0), (0, 16 - C * 3)))
    xyz_t = _xyz.T
    chol_t = (_cholesky + cholesky_bound).T

    grid = (H * W // PIX_BLK, N // N_BLK)
    out_flat = pl.pallas_call(
        _raster_body,
        grid=grid,
        in_specs=[
            pl.BlockSpec((2, N_BLK), lambda p, n: (0, n)),
            pl.BlockSpec((3, N_BLK), lambda p, n: (0, n)),
            pl.BlockSpec((N_BLK, 16), lambda p, n: (n, 0)),
        ],
        out_specs=pl.BlockSpec((PIX_BLK, 16), lambda p, n: (p, 0)),
        out_shape=jax.ShapeDtypeStruct((H * W, 16), jnp.float32),
        compiler_params=pltpu.CompilerParams(
            dimension_semantics=("parallel", "arbitrary"),
        ),
    )(xyz_t, chol_t, colors)

    out = out_flat[:, : C * 3].reshape(H, W, C, 3)
    return jnp.transpose(out, (2, 3, 0, 1))


# speedup vs baseline: 1.4288x; 1.4288x over previous
import functools

import jax
import jax.numpy as jnp
from jax import lax
from jax.experimental import pallas as pl
from jax.experimental.pallas import tpu as pltpu

N = 4096
C = 3
H = 256
W = 256

SROWS = 8
NSTRIP = H // SROWS
SPIX = SROWS * W
KB = 256
K5 = 5
MAXB = (K5 * N) // KB + NSTRIP
CAP = MAXB * KB
SIG_CUT = 14.0


def _project(xyz_t, chol_t):
    xy = jnp.tanh(xyz_t)
    cx = 0.5 * W * (xy[0] + 1.0)
    cy = 0.5 * H * (xy[1] + 1.0)
    l11, l21, l22 = chol_t[0], chol_t[1], chol_t[2]
    s11 = l11 * l11
    s12 = l11 * l21
    s22 = l21 * l21 + l22 * l22
    det = s11 * s22 - s12 * s12
    inv_det = 1.0 / det
    a = (0.5 * s22) * inv_det
    b = (-s12) * inv_det
    c = (0.5 * s11) * inv_det
    tr = s11 + s22
    lam_ub = tr - det / tr
    r2 = 2.0 * SIG_CUT * lam_ub
    return cx, cy, a, b, c, r2


def _bin_strips(cx, cy, a, b, c, r2, colors):
    r = jnp.sqrt(r2)
    slo = jnp.clip(jnp.floor((cy - r) / SROWS).astype(jnp.int32), 0, NSTRIP - 1)
    shi = jnp.clip(jnp.floor((cy + r) / SROWS).astype(jnp.int32), 0, NSTRIP - 1)

    gidx = jnp.tile(jnp.arange(N, dtype=jnp.int32), K5)
    kk = jnp.repeat(jnp.arange(K5, dtype=jnp.int32), N)
    s_k = slo[gidx] + kk
    active = s_k <= shi[gidx]
    key = jnp.where(active, s_k, NSTRIP)

    order = jnp.argsort(key, stable=True)
    ks = key[order]
    gs = gidx[order]

    count_s = jnp.bincount(key, length=NSTRIP + 1)[:NSTRIP].astype(jnp.int32)
    blocks_s = jnp.maximum(1, (count_s + KB - 1) // KB)
    bstart = jnp.concatenate([jnp.zeros(1, jnp.int32),
                              jnp.cumsum(blocks_s)[:-1].astype(jnp.int32)])
    nblocks = jnp.sum(blocks_s)
    rowstart = bstart * KB
    cstart = jnp.concatenate([jnp.zeros(1, jnp.int32),
                              jnp.cumsum(count_s)[:-1].astype(jnp.int32)])
    pos = jnp.arange(K5 * N, dtype=jnp.int32) - cstart[jnp.clip(ks, 0, NSTRIP - 1)]
    target = jnp.where(ks < NSTRIP, rowstart[jnp.clip(ks, 0, NSTRIP - 1)] + pos, CAP)

    params = jnp.stack([cx, cy, a, b, c,
                        jnp.zeros_like(a), jnp.zeros_like(a), jnp.zeros_like(a)])
    geom_s = jnp.zeros((8, CAP + 8), jnp.float32).at[:, target].set(params[:, gs])
    colors_s = jnp.zeros((CAP + 8, 16), jnp.float32).at[target].set(colors[gs])

    ii = jnp.arange(MAXB, dtype=jnp.int32)
    ws = jnp.searchsorted(jnp.cumsum(blocks_s), ii, side="right").astype(jnp.int32)
    valid = (ii < nblocks).astype(jnp.int32)
    ws = jnp.where(valid == 1, ws, 0)
    isf = jnp.zeros(MAXB, jnp.int32).at[bstart].set(1) * valid
    return geom_s, colors_s, ws, isf, valid


def _raster_body(ws_ref, isf_ref, val_ref, geom_ref, col_ref, out_ref):
    i = pl.program_id(0)

    @pl.when(val_ref[i] == 1)
    def _():
        cx = geom_ref[0:1, :]
        cy = geom_ref[1:2, :]
        a = geom_ref[2:3, :]
        b = geom_ref[3:4, :]
        c = geom_ref[4:5, :]

        lin = lax.broadcasted_iota(jnp.int32, (SPIX, 1), 0)
        row0 = (ws_ref[i] * SROWS).astype(jnp.float32)
        xs = (lin % W).astype(jnp.float32) + 0.5
        ys = (lin // W).astype(jnp.float32) + (row0 + 0.5)

        dx = xs - cx
        dy = ys - cy
        sigma = (a * dx + b * dy) * dx + c * (dy * dy)
        wgt = jnp.exp(-jnp.maximum(sigma, 0.0))
        contrib = jnp.dot(wgt, col_ref[...], preferred_element_type=jnp.float32)
        out_ref[...] = jnp.where(isf_ref[i] == 1, contrib, out_ref[...] + contrib)


def kernel(_xyz, _cholesky, _features_dc, _opacity, cholesky_bound, render_colors):
    colors = jnp.transpose(_features_dc, (1, 0, 2)).reshape(N, C * 3)
    colors = colors * _opacity[:, 0:1]
    colors = jnp.pad(colors, ((0, 0), (0, 16 - C * 3)))
    xyz_t = _xyz.T
    chol_t = (_cholesky + cholesky_bound).T

    cx, cy, a, b, c, r2 = _project(xyz_t, chol_t)
    geom_s, colors_s, ws, isf, valid = _bin_strips(cx, cy, a, b, c, r2, colors)

    grid_spec = pltpu.PrefetchScalarGridSpec(
        num_scalar_prefetch=3,
        grid=(MAXB,),
        in_specs=[
            pl.BlockSpec((8, KB), lambda i, ws, isf, val: (0, i)),
            pl.BlockSpec((KB, 16), lambda i, ws, isf, val: (i, 0)),
        ],
        out_specs=pl.BlockSpec((SPIX, 16), lambda i, ws, isf, val: (ws[i], 0)),
    )
    out_flat = pl.pallas_call(
        _raster_body,
        grid_spec=grid_spec,
        out_shape=jax.ShapeDtypeStruct((H * W, 16), jnp.float32),
        compiler_params=pltpu.CompilerParams(
            dimension_semantics=("arbitrary",),
        ),
    )(ws, isf, valid, geom_s, colors_s)

    out = out_flat[:, : C * 3].reshape(H, W, C, 3)
    return jnp.transpose(out, (2, 3, 0, 1))


# speedup vs baseline: 1.4289x; 1.0001x over previous
import functools

import jax
import jax.numpy as jnp
from jax import lax
from jax.experimental import pallas as pl
from jax.experimental.pallas import tpu as pltpu

N = 4096
C = 3
H = 256
W = 256

SROWS = 8
NSTRIP = H // SROWS
SPIX = SROWS * W
KB = 256
K5 = 5
MAXB = (K5 * N) // KB + NSTRIP
CAP = MAXB * KB
SIG_CUT = 14.0


def _project(xyz_t, chol_t):
    xy = jnp.tanh(xyz_t)
    cx = 0.5 * W * (xy[0] + 1.0)
    cy = 0.5 * H * (xy[1] + 1.0)
    l11, l21, l22 = chol_t[0], chol_t[1], chol_t[2]
    s11 = l11 * l11
    s12 = l11 * l21
    s22 = l21 * l21 + l22 * l22
    det = s11 * s22 - s12 * s12
    inv_det = 1.0 / det
    a = (0.5 * s22) * inv_det
    b = (-s12) * inv_det
    c = (0.5 * s11) * inv_det
    tr = s11 + s22
    lam_ub = tr - det / tr
    r2 = 2.0 * SIG_CUT * lam_ub
    return cx, cy, a, b, c, r2


def _bin_strips(cx, cy, a, b, c, r2, colors):
    r = jnp.sqrt(r2)
    slo = jnp.clip(jnp.floor((cy - r) / SROWS).astype(jnp.int32), 0, NSTRIP - 1)
    shi = jnp.clip(jnp.floor((cy + r) / SROWS).astype(jnp.int32), 0, NSTRIP - 1)

    gidx = jnp.tile(jnp.arange(N, dtype=jnp.int32), K5)
    kk = jnp.repeat(jnp.arange(K5, dtype=jnp.int32), N)
    s_k = slo[gidx] + kk
    active = s_k <= shi[gidx]
    key = jnp.where(active, s_k, NSTRIP)

    order = jnp.argsort(key, stable=True)
    ks = key[order]
    gs = gidx[order]

    count_s = jnp.bincount(key, length=NSTRIP + 1)[:NSTRIP].astype(jnp.int32)
    blocks_s = jnp.maximum(1, (count_s + KB - 1) // KB)
    bstart = jnp.concatenate([jnp.zeros(1, jnp.int32),
                              jnp.cumsum(blocks_s)[:-1].astype(jnp.int32)])
    nblocks = jnp.sum(blocks_s)
    rowstart = bstart * KB
    cstart = jnp.concatenate([jnp.zeros(1, jnp.int32),
                              jnp.cumsum(count_s)[:-1].astype(jnp.int32)])
    pos = jnp.arange(K5 * N, dtype=jnp.int32) - cstart[jnp.clip(ks, 0, NSTRIP - 1)]
    target = jnp.where(ks < NSTRIP, rowstart[jnp.clip(ks, 0, NSTRIP - 1)] + pos, CAP)

    params = jnp.stack([cx, cy, a, b, c,
                        jnp.zeros_like(a), jnp.zeros_like(a), jnp.zeros_like(a)])
    geom_s = jnp.zeros((8, CAP + 8), jnp.float32).at[:, target].set(params[:, gs])
    colors_s = jnp.zeros((CAP + 8, 16), jnp.float32).at[target].set(colors[gs])

    ii = jnp.arange(MAXB, dtype=jnp.int32)
    ws = jnp.searchsorted(jnp.cumsum(blocks_s), ii, side="right").astype(jnp.int32)
    valid = (ii < nblocks).astype(jnp.int32)
    ws = jnp.where(valid == 1, ws, NSTRIP - 1)
    isf = jnp.zeros(MAXB, jnp.int32).at[bstart].set(1) * valid
    return geom_s, colors_s, ws, isf, valid


def _raster_body(ws_ref, isf_ref, val_ref, geom_ref, col_ref, out_ref):
    i = pl.program_id(0)

    @pl.when(val_ref[i] == 1)
    def _():
        cx = geom_ref[0:1, :]
        cy = geom_ref[1:2, :]
        a = geom_ref[2:3, :]
        b = geom_ref[3:4, :]
        c = geom_ref[4:5, :]

        lin = lax.broadcasted_iota(jnp.int32, (SPIX, 1), 0)
        row0 = (ws_ref[i] * SROWS).astype(jnp.float32)
        xs = (lin % W).astype(jnp.float32) + 0.5
        ys = (lin // W).astype(jnp.float32) + (row0 + 0.5)

        dx = xs - cx
        dy = ys - cy
        sigma = (a * dx + b * dy) * dx + c * (dy * dy)
        wgt = jnp.exp(-jnp.maximum(sigma, 0.0))
        contrib = jnp.dot(wgt, col_ref[...], preferred_element_type=jnp.float32)
        out_ref[...] = jnp.where(isf_ref[i] == 1, contrib, out_ref[...] + contrib)


def kernel(_xyz, _cholesky, _features_dc, _opacity, cholesky_bound, render_colors):
    colors = jnp.transpose(_features_dc, (1, 0, 2)).reshape(N, C * 3)
    colors = colors * _opacity[:, 0:1]
    colors = jnp.pad(colors, ((0, 0), (0, 16 - C * 3)))
    xyz_t = _xyz.T
    chol_t = (_cholesky + cholesky_bound).T

    cx, cy, a, b, c, r2 = _project(xyz_t, chol_t)
    geom_s, colors_s, ws, isf, valid = _bin_strips(cx, cy, a, b, c, r2, colors)

    grid_spec = pltpu.PrefetchScalarGridSpec(
        num_scalar_prefetch=3,
        grid=(MAXB,),
        in_specs=[
            pl.BlockSpec((8, KB), lambda i, ws, isf, val: (0, i)),
            pl.BlockSpec((KB, 16), lambda i, ws, isf, val: (i, 0)),
        ],
        out_specs=pl.BlockSpec((SPIX, 16), lambda i, ws, isf, val: (ws[i], 0)),
    )
    out_flat = pl.pallas_call(
        _raster_body,
        grid_spec=grid_spec,
        out_shape=jax.ShapeDtypeStruct((H * W, 16), jnp.float32),
        compiler_params=pltpu.CompilerParams(
            dimension_semantics=("arbitrary",),
        ),
    )(ws, isf, valid, geom_s, colors_s)

    out = out_flat[:, : C * 3].reshape(H, W, C, 3)
    return jnp.transpose(out, (2, 3, 0, 1))


# speedup vs baseline: 2.9910x; 2.0932x over previous
import functools

import jax
import jax.numpy as jnp
from jax import lax
from jax.experimental import pallas as pl
from jax.experimental.pallas import tpu as pltpu
from jax.experimental.pallas import tpu_sc as plsc

N = 4096
C = 3
H = 256
W = 256

SROWS = 8
NSTRIP = H // SROWS
SPIX = SROWS * W
KB = 256
K5 = 5
MAXB = (K5 * N) // KB + NSTRIP
CAP = MAXB * KB
SIG_CUT = 14.0

NW = 32
RPW = CAP // NW
RCH = RPW // 128


def _conic(l11, l21, l22):
    s11 = l11 * l11
    s12 = l11 * l21
    s22 = l21 * l21 + l22 * l22
    det = s11 * s22 - s12 * s12
    inv_det = 1.0 / det
    a = (0.5 * s22) * inv_det
    b = (-s12) * inv_det
    c = (0.5 * s11) * inv_det
    tr = s11 + s22
    lam_ub = tr - det / tr
    r2 = 2.0 * SIG_CUT * lam_ub
    return a, b, c, r2


def _schedule(cy, r2):
    r = jnp.sqrt(r2)
    slo = jnp.clip(jnp.floor((cy - r) / SROWS).astype(jnp.int32), 0, NSTRIP - 1)
    shi = jnp.clip(jnp.floor((cy + r) / SROWS).astype(jnp.int32), 0, NSTRIP - 1)

    sgrid = jnp.arange(NSTRIP, dtype=jnp.int32)
    indi = ((sgrid[None, :] >= slo[:, None]) &
            (sgrid[None, :] <= shi[:, None])).astype(jnp.int32)
    count_s = jnp.sum(indi, axis=0)
    exc = jnp.cumsum(indi, axis=0) - indi

    blocks_s = jnp.maximum(1, (count_s + KB - 1) // KB)
    cumb = jnp.cumsum(blocks_s)
    nblocks = cumb[-1]
    bstart = cumb - blocks_s
    rowstart = (bstart * KB).astype(jnp.int32)

    ks = jnp.arange(K5, dtype=jnp.int32)
    s_k = slo[:, None] + ks[None, :]
    act = s_k <= shi[:, None]
    s_kc = jnp.clip(s_k, 0, NSTRIP - 1)
    rank = jnp.take_along_axis(exc, s_kc, axis=1)
    tgt = jnp.where(act, rowstart[s_kc] + rank, CAP)

    gidx = jnp.broadcast_to(jnp.arange(N, dtype=jnp.int32)[:, None], (N, K5))
    src = jnp.full(CAP + 8, N, jnp.int32)
    src = src.at[tgt.reshape(-1)].set(gidx.reshape(-1))[:CAP]

    ii = jnp.arange(MAXB, dtype=jnp.int32)
    ws = jnp.searchsorted(cumb, ii, side="right").astype(jnp.int32)
    valid = (ii < nblocks).astype(jnp.int32)
    ws = jnp.where(valid == 1, ws, NSTRIP - 1)
    isf = jnp.zeros(MAXB, jnp.int32).at[bstart].set(1) * valid
    return src, ws, isf, valid


def _sc_compact(x0, x1, l11a, l21a, l22a, cols9, src):
    mesh = plsc.VectorSubcoreMesh(core_axis_name="c", subcore_axis_name="s")

    @functools.partial(
        pl.kernel,
        mesh=mesh,
        out_type=(
            jax.ShapeDtypeStruct((8 * CAP,), jnp.float32),
            jax.ShapeDtypeStruct((16 * CAP,), jnp.float32),
        ),
        scratch_types=[
            pltpu.VMEM((RPW,), jnp.int32),
            pltpu.VMEM((RPW,), jnp.float32),
            pltpu.VMEM((RPW,), jnp.float32),
            pltpu.VMEM((RPW,), jnp.float32),
            pltpu.VMEM((RPW,), jnp.float32),
            pltpu.VMEM((RPW,), jnp.float32),
            pltpu.VMEM((RPW,), jnp.float32),
            pltpu.VMEM((RPW,), jnp.float32),
            pltpu.VMEM((RPW,), jnp.float32),
            pltpu.VMEM((RPW,), jnp.float32),
            pltpu.VMEM((RPW,), jnp.float32),
            pltpu.VMEM((RPW,), jnp.float32),
            pltpu.VMEM((RPW,), jnp.float32),
            pltpu.VMEM((RPW,), jnp.float32),
            pltpu.VMEM((RPW,), jnp.float32),
            pltpu.VMEM((RPW,), jnp.float32),
            pltpu.VMEM((RPW,), jnp.float32),
            pltpu.VMEM((RPW,), jnp.float32),
            pltpu.VMEM((RPW,), jnp.float32),
            pltpu.VMEM((RPW,), jnp.float32),
            pltpu.SemaphoreType.DMA,
        ],
    )
    def k(x0_hbm, x1_hbm, l11_hbm, l21_hbm, l22_hbm,
          c0_hbm, c1_hbm, c2_hbm, c3_hbm, c4_hbm, c5_hbm, c6_hbm, c7_hbm,
          c8_hbm, src_hbm,
          geom_hbm, csort_hbm,
          idx_v, p0_v, p1_v, p2_v, p3_v, p4_v,
          g0_v, g1_v, g2_v, g3_v, g4_v,
          o0_v, o1_v, o2_v, o3_v, o4_v, o5_v, o6_v, o7_v, o8_v, sem):
        cid = lax.axis_index("c")
        sid = lax.axis_index("s")
        w = sid * 2 + cid

        pltpu.sync_copy(src_hbm.at[pl.ds(w * RPW, RPW)], idx_v)

        copies = []
        for t in range(RCH):
            dd = pl.ds(t * 128, 128)
            ix = idx_v.at[dd]
            copies.append(pltpu.async_copy(x0_hbm.at[ix], p0_v.at[dd], sem))
            copies.append(pltpu.async_copy(x1_hbm.at[ix], p1_v.at[dd], sem))
            copies.append(pltpu.async_copy(l11_hbm.at[ix], p2_v.at[dd], sem))
            copies.append(pltpu.async_copy(l21_hbm.at[ix], p3_v.at[dd], sem))
            copies.append(pltpu.async_copy(l22_hbm.at[ix], p4_v.at[dd], sem))
            copies.append(pltpu.async_copy(c0_hbm.at[ix], o0_v.at[dd], sem))
            copies.append(pltpu.async_copy(c1_hbm.at[ix], o1_v.at[dd], sem))
            copies.append(pltpu.async_copy(c2_hbm.at[ix], o2_v.at[dd], sem))
            copies.append(pltpu.async_copy(c3_hbm.at[ix], o3_v.at[dd], sem))
            copies.append(pltpu.async_copy(c4_hbm.at[ix], o4_v.at[dd], sem))
            copies.append(pltpu.async_copy(c5_hbm.at[ix], o5_v.at[dd], sem))
            copies.append(pltpu.async_copy(c6_hbm.at[ix], o6_v.at[dd], sem))
            copies.append(pltpu.async_copy(c7_hbm.at[ix], o7_v.at[dd], sem))
            copies.append(pltpu.async_copy(c8_hbm.at[ix], o8_v.at[dd], sem))
        for cp in copies:
            cp.wait()

        for g in range(RPW // 16):
            dd = pl.ds(g * 16, 16)
            e2x = jnp.exp(2.0 * p0_v[dd])
            e2y = jnp.exp(2.0 * p1_v[dd])
            tx = 1.0 - 2.0 / (e2x + 1.0)
            ty = 1.0 - 2.0 / (e2y + 1.0)
            a, b, c, _ = _conic(p2_v[dd], p3_v[dd], p4_v[dd])
            g0_v[dd] = 0.5 * W * (tx + 1.0)
            g1_v[dd] = 0.5 * H * (ty + 1.0)
            g2_v[dd] = a
            g3_v[dd] = b
            g4_v[dd] = c

        pltpu.sync_copy(g0_v, geom_hbm.at[pl.ds(0 * CAP + w * RPW, RPW)])
        pltpu.sync_copy(g1_v, geom_hbm.at[pl.ds(1 * CAP + w * RPW, RPW)])
        pltpu.sync_copy(g2_v, geom_hbm.at[pl.ds(2 * CAP + w * RPW, RPW)])
        pltpu.sync_copy(g3_v, geom_hbm.at[pl.ds(3 * CAP + w * RPW, RPW)])
        pltpu.sync_copy(g4_v, geom_hbm.at[pl.ds(4 * CAP + w * RPW, RPW)])
        for j, ov in enumerate([o0_v, o1_v, o2_v, o3_v, o4_v,
                                o5_v, o6_v, o7_v, o8_v]):
            pltpu.sync_copy(ov, csort_hbm.at[pl.ds(j * CAP + w * RPW, RPW)])

    return k(x0, x1, l11a, l21a, l22a, *cols9, src)


def _raster_body(ws_ref, isf_ref, val_ref, geom_ref, col_ref, out_ref):
    i = pl.program_id(0)

    @pl.when(val_ref[i] == 1)
    def _():
        cx = geom_ref[0:1, :]
        cy = geom_ref[1:2, :]
        a = geom_ref[2:3, :]
        b = geom_ref[3:4, :]
        c = geom_ref[4:5, :]

        lin = lax.broadcasted_iota(jnp.int32, (SPIX, 1), 0)
        row0 = (ws_ref[i] * SROWS).astype(jnp.float32)
        xs = (lin % W).astype(jnp.float32) + 0.5
        ys = (lin // W).astype(jnp.float32) + (row0 + 0.5)

        dx = xs - cx
        dy = ys - cy
        sigma = (a * dx + b * dy) * dx + c * (dy * dy)
        sig_safe = jnp.where(sigma >= 0.0, jnp.minimum(sigma, 100.0), 0.0)
        wgt = jnp.exp(-sig_safe)
        contrib = lax.dot_general(
            wgt, col_ref[...], (((1,), (1,)), ((), ())),
            preferred_element_type=jnp.float32)
        out_ref[...] = jnp.where(isf_ref[i] == 1, contrib, out_ref[...] + contrib)


def kernel(_xyz, _cholesky, _features_dc, _opacity, cholesky_bound, render_colors):
    colors = jnp.transpose(_features_dc, (1, 0, 2)).reshape(N, C * 3)
    colors = colors * _opacity[:, 0:1]
    colsT = jnp.pad(colors, ((0, 8), (0, 0))).T
    cols9 = [colsT[j] for j in range(C * 3)]
    chol = _cholesky + cholesky_bound
    x0 = jnp.pad(_xyz[:, 0], (0, 8))
    x1 = jnp.pad(_xyz[:, 1], (0, 8))
    l11a = jnp.pad(chol[:, 0], (0, 8), constant_values=1.0)
    l21a = jnp.pad(chol[:, 1], (0, 8), constant_values=0.0)
    l22a = jnp.pad(chol[:, 2], (0, 8), constant_values=1.0)

    cy = 0.5 * H * (jnp.tanh(x1[:N]) + 1.0)
    _, _, _, r2 = _conic(l11a[:N], l21a[:N], l22a[:N])
    src, ws, isf, valid = _schedule(cy, r2)

    geom_flat, colt_flat = _sc_compact(x0, x1, l11a, l21a, l22a, cols9, src)
    geom_s = geom_flat.reshape(8, CAP)
    colors_s = colt_flat.reshape(16, CAP)

    grid_spec = pltpu.PrefetchScalarGridSpec(
        num_scalar_prefetch=3,
        grid=(MAXB,),
        in_specs=[
            pl.BlockSpec((8, KB), lambda i, ws, isf, val: (0, i)),
            pl.BlockSpec((16, KB), lambda i, ws, isf, val: (0, i)),
        ],
        out_specs=pl.BlockSpec((SPIX, 16), lambda i, ws, isf, val: (ws[i], 0)),
    )
    out_flat = pl.pallas_call(
        _raster_body,
        grid_spec=grid_spec,
        out_shape=jax.ShapeDtypeStruct((H * W, 16), jnp.float32),
        compiler_params=pltpu.CompilerParams(
            dimension_semantics=("arbitrary",),
        ),
    )(ws, isf, valid, geom_s, colors_s)

    out = out_flat[:, : C * 3].reshape(H, W, C, 3)
    return jnp.transpose(out, (2, 3, 0, 1))
